# baseline (speedup 1.0000x reference)
import functools

import jax
import jax.numpy as jnp
from jax.experimental import pallas as pl
from jax.experimental.pallas import tpu as pltpu

IMG_H, IMG_W = 800, 800
OUT = 14
P = 8
NPIX = (OUT // 2) * (OUT // 2)
NCOLP = 4 * NPIX
NCOL = P * NCOLP
WFP = 64


def _roi_kernel(props_ref, propsn_ref, feat_ref, out_ref, w2_ref, *, hf, wf):
    feat = feat_ref[0]
    fx = jnp.float32(hf / IMG_H)
    fy = jnp.float32(wf / IMG_W)
    one = jnp.float32(1.0)
    bone = jnp.bfloat16(1.0)
    bzero = jnp.bfloat16(0.0)

    col = jax.lax.broadcasted_iota(jnp.int32, (1, NCOL), 1)
    pcol = col // NCOLP
    r196 = col % NCOLP
    q, ij = r196 // NPIX, r196 % NPIX
    ii = 2 * (ij // (OUT // 2)) + q // 2
    jj = 2 * (ij % (OUT // 2)) + q % 2
    ti = ii.astype(jnp.float32) / jnp.float32(OUT - 1)
    tj = jj.astype(jnp.float32) / jnp.float32(OUT - 1)
    rowx = jax.lax.broadcasted_iota(jnp.int32, (hf, NCOL), 0).astype(jnp.float32)
    rowy = jax.lax.broadcasted_iota(jnp.int32, (WFP, NCOL), 0).astype(jnp.float32)

    def build_w(pref):
        zero = jnp.zeros((1, 1), jnp.float32)
        x0v, y0v, wv, hv = zero, zero, zero, zero
        for p in range(P):
            sel = pcol == p
            x0v = jnp.where(sel, pref[0, p : p + 1, 0:1], x0v)
            y0v = jnp.where(sel, pref[0, p : p + 1, 1:2], y0v)
            wv = jnp.where(sel, pref[0, p : p + 1, 2:3], wv)
            hv = jnp.where(sel, pref[0, p : p + 1, 3:4], hv)
        x0v = jnp.floor(x0v * fx)
        y0v = jnp.floor(y0v * fy)
        wv = jnp.ceil(wv * fx)
        hv = jnp.ceil(hv * fy)
        sx = jnp.clip(x0v + ti * (wv - one), 0.0, float(hf - 1))
        sy = jnp.clip(y0v + tj * (hv - one), 0.0, float(wf - 1))
        xd = jnp.abs(rowx - sx).astype(jnp.bfloat16)
        yd = jnp.abs(rowy - sy).astype(jnp.bfloat16)
        xt = jnp.maximum(bone - xd, bzero)
        yt = jnp.maximum(bone - yd, bzero)
        xt_full = jnp.broadcast_to(xt[:, None, :], (hf, WFP, NCOL))
        yt_full = jnp.broadcast_to(yt[None, :, :], (hf, WFP, NCOL))
        return (xt_full * yt_full).reshape(hf * WFP, NCOL)

    def dot_pool(buf):
        o = jnp.dot(feat, w2_ref[buf], preferred_element_type=jnp.float32)
        for p in range(P):
            base = p * NCOLP
            m0 = jnp.maximum(o[:, base : base + NPIX],
                             o[:, base + NPIX : base + 2 * NPIX])
            m1 = jnp.maximum(o[:, base + 2 * NPIX : base + 3 * NPIX],
                             o[:, base + 3 * NPIX : base + 4 * NPIX])
            out_ref[0, p] = jnp.maximum(m0, m1)

    nn = pl.program_id(1)

    @pl.when(nn == 0)
    def _():
        w2_ref[0] = build_w(props_ref)

    @pl.when(nn % 2 == 0)
    def _():
        w2_ref[1] = build_w(propsn_ref)
        dot_pool(0)

    @pl.when(nn % 2 == 1)
    def _():
        w2_ref[0] = build_w(propsn_ref)
        dot_pool(1)


@jax.jit
def kernel(proposals, features):
    b, c, hf, wf = features.shape
    n = proposals.shape[1]
    nsteps = n // P
    featp = jnp.pad(features.astype(jnp.bfloat16),
                    ((0, 0), (0, 0), (0, 0), (0, WFP - wf)))
    feat2 = featp.reshape(b, c, hf * WFP)

    out = pl.pallas_call(
        functools.partial(_roi_kernel, hf=hf, wf=wf),
        grid=(b, nsteps),
        in_specs=[
            pl.BlockSpec((1, P, 4), lambda bb, nn: (bb, nn, 0)),
            pl.BlockSpec((1, P, 4),
                         lambda bb, nn: (bb, jnp.minimum(nn + 1, nsteps - 1), 0)),
            pl.BlockSpec((1, c, hf * WFP), lambda bb, nn: (bb, 0, 0)),
        ],
        out_specs=pl.BlockSpec((1, P, c, NPIX), lambda bb, nn: (bb, nn, 0, 0)),
        out_shape=jax.ShapeDtypeStruct((b, n, c, NPIX), jnp.float32),
        scratch_shapes=[pltpu.VMEM((2, hf * WFP, NCOL), jnp.bfloat16)],
        compiler_params=pltpu.CompilerParams(
            dimension_semantics=("arbitrary", "arbitrary"),
            vmem_limit_bytes=100 * 1024 * 1024,
        ),
    )(proposals, proposals, feat2)
    return out.reshape(b, n, c, OUT // 2, OUT // 2)


# speedup vs baseline: 1.6126x; 1.6126x over previous
import functools

import jax
import jax.numpy as jnp
from jax.experimental import pallas as pl
from jax.experimental.pallas import tpu as pltpu

IMG_H, IMG_W = 800, 800
OUT = 14
P = 16
NPIX = (OUT // 2) * (OUT // 2)
NCOLP = 4 * NPIX
NCOL = P * NCOLP
WFP = 64


def _roi_kernel(props_ref, feat_ref, out_ref, *, hf, wf):
    feat = feat_ref[0]
    fx = jnp.float32(hf / IMG_H)
    fy = jnp.float32(wf / IMG_W)
    one = jnp.float32(1.0)
    bone = jnp.bfloat16(1.0)
    bzero = jnp.bfloat16(0.0)

    col = jax.lax.broadcasted_iota(jnp.int32, (1, NCOL), 1)
    pcol = col // NCOLP
    r196 = col % NCOLP
    q, ij = r196 // NPIX, r196 % NPIX
    ii = 2 * (ij // (OUT // 2)) + q // 2
    jj = 2 * (ij % (OUT // 2)) + q % 2
    ti = ii.astype(jnp.float32) / jnp.float32(OUT - 1)
    tj = jj.astype(jnp.float32) / jnp.float32(OUT - 1)
    rowx = jax.lax.broadcasted_iota(jnp.int32, (hf, NCOL), 0).astype(jnp.float32)
    rowy = jax.lax.broadcasted_iota(jnp.int32, (WFP, NCOL), 0).astype(jnp.float32)

    zero = jnp.zeros((1, 1), jnp.float32)
    x0v, y0v, wv, hv = zero, zero, zero, zero
    for p in range(P):
        sel = pcol == p
        x0v = jnp.where(sel, props_ref[0, p : p + 1, 0:1], x0v)
        y0v = jnp.where(sel, props_ref[0, p : p + 1, 1:2], y0v)
        wv = jnp.where(sel, props_ref[0, p : p + 1, 2:3], wv)
        hv = jnp.where(sel, props_ref[0, p : p + 1, 3:4], hv)
    x0v = jnp.floor(x0v * fx)
    y0v = jnp.floor(y0v * fy)
    wv = jnp.ceil(wv * fx)
    hv = jnp.ceil(hv * fy)
    sx = jnp.clip(x0v + ti * (wv - one), 0.0, float(hf - 1))
    sy = jnp.clip(y0v + tj * (hv - one), 0.0, float(wf - 1))

    xd = jnp.abs(rowx - sx).astype(jnp.bfloat16)
    yd = jnp.abs(rowy - sy).astype(jnp.bfloat16)
    xt = jnp.maximum(bone - xd, bzero)
    yt = jnp.maximum(bone - yd, bzero)
    xt_full = jnp.broadcast_to(xt[:, None, :], (hf, WFP, NCOL))
    yt_full = jnp.broadcast_to(yt[None, :, :], (hf, WFP, NCOL))
    w = (xt_full * yt_full).reshape(hf * WFP, NCOL)

    o = jnp.dot(feat, w, preferred_element_type=jnp.float32)
    for p in range(P):
        base = p * NCOLP
        m0 = jnp.maximum(o[:, base : base + NPIX],
                         o[:, base + NPIX : base + 2 * NPIX])
        m1 = jnp.maximum(o[:, base + 2 * NPIX : base + 3 * NPIX],
                         o[:, base + 3 * NPIX : base + 4 * NPIX])
        out_ref[0, p] = jnp.maximum(m0, m1)


@jax.jit
def kernel(proposals, features):
    b, c, hf, wf = features.shape
    n = proposals.shape[1]
    featp = jnp.pad(features.astype(jnp.bfloat16),
                    ((0, 0), (0, 0), (0, 0), (0, WFP - wf)))
    feat2 = featp.reshape(b, c, hf * WFP)

    out = pl.pallas_call(
        functools.partial(_roi_kernel, hf=hf, wf=wf),
        grid=(b, n // P),
        in_specs=[
            pl.BlockSpec((1, P, 4), lambda bb, nn: (bb, nn, 0)),
            pl.BlockSpec((1, c, hf * WFP), lambda bb, nn: (bb, 0, 0)),
        ],
        out_specs=pl.BlockSpec((1, P, c, NPIX), lambda bb, nn: (bb, nn, 0, 0)),
        out_shape=jax.ShapeDtypeStruct((b, n, c, NPIX), jnp.float32),
        compiler_params=pltpu.CompilerParams(
            dimension_semantics=("arbitrary", "arbitrary"),
            vmem_limit_bytes=100 * 1024 * 1024,
        ),
    )(proposals, feat2)
    return out.reshape(b, n, c, OUT // 2, OUT // 2)


# speedup vs baseline: 1.6670x; 1.0337x over previous
import functools

import jax
import jax.numpy as jnp
from jax.experimental import pallas as pl
from jax.experimental.pallas import tpu as pltpu

IMG_H, IMG_W = 800, 800
OUT = 14
P = 16
NPIX = (OUT // 2) * (OUT // 2)
NCOLP = 4 * NPIX
NCOL = P * NCOLP
WFP = 64


def _roi_kernel(props_ref, feat_ref, out_ref, *, hf, wf):
    feat = feat_ref[0]
    fx = jnp.float32(hf / IMG_H)
    fy = jnp.float32(wf / IMG_W)
    one = jnp.float32(1.0)
    bone = jnp.bfloat16(1.0)
    bzero = jnp.bfloat16(0.0)

    col = jax.lax.broadcasted_iota(jnp.int32, (1, NCOL), 1)
    pcol = col // NCOLP
    r196 = col % NCOLP
    q, ij = r196 // NPIX, r196 % NPIX
    ii = 2 * (ij // (OUT // 2)) + q // 2
    jj = 2 * (ij % (OUT // 2)) + q % 2
    ti = ii.astype(jnp.float32) / jnp.float32(OUT - 1)
    tj = jj.astype(jnp.float32) / jnp.float32(OUT - 1)
    rowx = jax.lax.broadcasted_iota(jnp.int32, (hf, NCOL), 0).astype(jnp.float32)
    rowy = jax.lax.broadcasted_iota(jnp.int32, (WFP, NCOL), 0).astype(jnp.float32)

    zero = jnp.zeros((1, 1), jnp.float32)
    x0v, y0v, wv, hv = zero, zero, zero, zero
    for p in range(P):
        sel = pcol == p
        x0v = jnp.where(sel, props_ref[0, p : p + 1, 0:1], x0v)
        y0v = jnp.where(sel, props_ref[0, p : p + 1, 1:2], y0v)
        wv = jnp.where(sel, props_ref[0, p : p + 1, 2:3], wv)
        hv = jnp.where(sel, props_ref[0, p : p + 1, 3:4], hv)
    x0v = jnp.floor(x0v * fx)
    y0v = jnp.floor(y0v * fy)
    wv = jnp.ceil(wv * fx)
    hv = jnp.ceil(hv * fy)
    sx = jnp.clip(x0v + ti * (wv - one), 0.0, float(hf - 1))
    sy = jnp.clip(y0v + tj * (hv - one), 0.0, float(wf - 1))

    xd = jnp.abs(rowx - sx).astype(jnp.bfloat16)
    yd = jnp.abs(rowy - sy).astype(jnp.bfloat16)
    xt = jnp.maximum(bone - xd, bzero)
    yt = jnp.maximum(bone - yd, bzero)
    xt_full = jnp.broadcast_to(xt[:, None, :], (hf, WFP, NCOL))
    yt_full = jnp.broadcast_to(yt[None, :, :], (hf, WFP, NCOL))
    w = (xt_full * yt_full).reshape(hf * WFP, NCOL)

    o = jnp.dot(feat, w, preferred_element_type=jnp.float32).astype(jnp.bfloat16)
    for p in range(P):
        base = p * NCOLP
        m0 = jnp.maximum(o[:, base : base + NPIX],
                         o[:, base + NPIX : base + 2 * NPIX])
        m1 = jnp.maximum(o[:, base + 2 * NPIX : base + 3 * NPIX],
                         o[:, base + 3 * NPIX : base + 4 * NPIX])
        out_ref[0, p] = jnp.maximum(m0, m1)


@jax.jit
def kernel(proposals, features):
    b, c, hf, wf = features.shape
    n = proposals.shape[1]
    featp = jnp.pad(features.astype(jnp.bfloat16),
                    ((0, 0), (0, 0), (0, 0), (0, WFP - wf)))
    feat2 = featp.reshape(b, c, hf * WFP)

    out = pl.pallas_call(
        functools.partial(_roi_kernel, hf=hf, wf=wf),
        grid=(b, n // P),
        in_specs=[
            pl.BlockSpec((1, P, 4), lambda bb, nn: (bb, nn, 0)),
            pl.BlockSpec((1, c, hf * WFP), lambda bb, nn: (bb, 0, 0)),
        ],
        out_specs=pl.BlockSpec((1, P, c, NPIX), lambda bb, nn: (bb, nn, 0, 0)),
        out_shape=jax.ShapeDtypeStruct((b, n, c, NPIX), jnp.bfloat16),
        compiler_params=pltpu.CompilerParams(
            dimension_semantics=("arbitrary", "arbitrary"),
            vmem_limit_bytes=100 * 1024 * 1024,
        ),
    )(proposals, feat2)
    return out.reshape(b, n, c, OUT // 2, OUT // 2).astype(jnp.float32)


# speedup vs baseline: 1.7303x; 1.0380x over previous
import functools

import jax
import jax.numpy as jnp
from jax.experimental import pallas as pl
from jax.experimental.pallas import tpu as pltpu

IMG_H, IMG_W = 800, 800
OUT = 14
P = 32
NPIX = (OUT // 2) * (OUT // 2)
NCOLP = 4 * NPIX
NCOL = P * NCOLP
WFP = 64


def _roi_kernel(props_ref, feat_ref, out_ref, *, hf, wf):
    feat = feat_ref[0]
    fx = jnp.float32(hf / IMG_H)
    fy = jnp.float32(wf / IMG_W)
    one = jnp.float32(1.0)
    bone = jnp.bfloat16(1.0)
    bzero = jnp.bfloat16(0.0)

    col = jax.lax.broadcasted_iota(jnp.int32, (1, NCOL), 1)
    pcol = col // NCOLP
    r196 = col % NCOLP
    q, ij = r196 // NPIX, r196 % NPIX
    ii = 2 * (ij // (OUT // 2)) + q // 2
    jj = 2 * (ij % (OUT // 2)) + q % 2
    ti = ii.astype(jnp.float32) / jnp.float32(OUT - 1)
    tj = jj.astype(jnp.float32) / jnp.float32(OUT - 1)
    rowx = jax.lax.broadcasted_iota(jnp.int32, (hf, NCOL), 0).astype(jnp.float32)
    rowy = jax.lax.broadcasted_iota(jnp.int32, (WFP, NCOL), 0).astype(jnp.float32)

    zero = jnp.zeros((1, 1), jnp.float32)
    x0v, y0v, wv, hv = zero, zero, zero, zero
    for p in range(P):
        sel = pcol == p
        x0v = jnp.where(sel, props_ref[0, p : p + 1, 0:1], x0v)
        y0v = jnp.where(sel, props_ref[0, p : p + 1, 1:2], y0v)
        wv = jnp.where(sel, props_ref[0, p : p + 1, 2:3], wv)
        hv = jnp.where(sel, props_ref[0, p : p + 1, 3:4], hv)
    x0v = jnp.floor(x0v * fx)
    y0v = jnp.floor(y0v * fy)
    wv = jnp.ceil(wv * fx)
    hv = jnp.ceil(hv * fy)
    sx = jnp.clip(x0v + ti * (wv - one), 0.0, float(hf - 1))
    sy = jnp.clip(y0v + tj * (hv - one), 0.0, float(wf - 1))

    xd = jnp.abs(rowx - sx).astype(jnp.bfloat16)
    yd = jnp.abs(rowy - sy).astype(jnp.bfloat16)
    xt = jnp.maximum(bone - xd, bzero)
    yt = jnp.maximum(bone - yd, bzero)
    xt_full = jnp.broadcast_to(xt[:, None, :], (hf, WFP, NCOL))
    yt_full = jnp.broadcast_to(yt[None, :, :], (hf, WFP, NCOL))
    w = (xt_full * yt_full).reshape(hf * WFP, NCOL)

    o = jnp.dot(feat, w, preferred_element_type=jnp.float32).astype(jnp.bfloat16)
    for p in range(P):
        base = p * NCOLP
        m0 = jnp.maximum(o[:, base : base + NPIX],
                         o[:, base + NPIX : base + 2 * NPIX])
        m1 = jnp.maximum(o[:, base + 2 * NPIX : base + 3 * NPIX],
                         o[:, base + 3 * NPIX : base + 4 * NPIX])
        out_ref[0, p] = jnp.maximum(m0, m1)


@jax.jit
def kernel(proposals, features):
    b, c, hf, wf = features.shape
    n = proposals.shape[1]
    featp = jnp.pad(features.astype(jnp.bfloat16),
                    ((0, 0), (0, 0), (0, 0), (0, WFP - wf)))
    feat2 = featp.reshape(b, c, hf * WFP)

    out = pl.pallas_call(
        functools.partial(_roi_kernel, hf=hf, wf=wf),
        grid=(b, n // P),
        in_specs=[
            pl.BlockSpec((1, P, 4), lambda bb, nn: (bb, nn, 0)),
            pl.BlockSpec((1, c, hf * WFP), lambda bb, nn: (bb, 0, 0)),
        ],
        out_specs=pl.BlockSpec((1, P, c, NPIX), lambda bb, nn: (bb, nn, 0, 0)),
        out_shape=jax.ShapeDtypeStruct((b, n, c, NPIX), jnp.bfloat16),
        compiler_params=pltpu.CompilerParams(
            dimension_semantics=("arbitrary", "arbitrary"),
            vmem_limit_bytes=100 * 1024 * 1024,
        ),
    )(proposals, feat2)
    return out.reshape(b, n, c, OUT // 2, OUT // 2).astype(jnp.float32)


# speedup vs baseline: 1.8012x; 1.0410x over previous
import functools

import numpy as np

import jax
import jax.numpy as jnp
from jax.experimental import pallas as pl
from jax.experimental.pallas import tpu as pltpu

IMG_H, IMG_W = 800, 800
OUT = 14
P = 32
NPIX = (OUT // 2) * (OUT // 2)
NCOLP = 4 * NPIX
NCOL = P * NCOLP
WFP = 64

_q = np.arange(NCOLP) // NPIX
_ij = np.arange(NCOLP) % NPIX
_TI = ((2 * (_ij // (OUT // 2)) + _q // 2) / (OUT - 1)).astype(np.float32)
_TJ = ((2 * (_ij % (OUT // 2)) + _q % 2) / (OUT - 1)).astype(np.float32)


def _roi_kernel(sx_ref, sy_ref, feat_ref, out_ref, *, hf, wf):
    feat = feat_ref[0]
    bone = jnp.bfloat16(1.0)
    bzero = jnp.bfloat16(0.0)

    sx = sx_ref[0, 0]
    sy = sy_ref[0, 0]
    rowx = jax.lax.broadcasted_iota(jnp.int32, (hf, NCOL), 0).astype(jnp.float32)
    rowy = jax.lax.broadcasted_iota(jnp.int32, (WFP, NCOL), 0).astype(jnp.float32)

    xd = jnp.abs(rowx - sx).astype(jnp.bfloat16)
    yd = jnp.abs(rowy - sy).astype(jnp.bfloat16)
    xt = jnp.maximum(bone - xd, bzero)
    yt = jnp.maximum(bone - yd, bzero)
    xt_full = jnp.broadcast_to(xt[:, None, :], (hf, WFP, NCOL))
    yt_full = jnp.broadcast_to(yt[None, :, :], (hf, WFP, NCOL))
    w = (xt_full * yt_full).reshape(hf * WFP, NCOL)

    o = jnp.dot(feat, w, preferred_element_type=jnp.float32).astype(jnp.bfloat16)
    for p in range(P):
        base = p * NCOLP
        m0 = jnp.maximum(o[:, base : base + NPIX],
                         o[:, base + NPIX : base + 2 * NPIX])
        m1 = jnp.maximum(o[:, base + 2 * NPIX : base + 3 * NPIX],
                         o[:, base + 3 * NPIX : base + 4 * NPIX])
        out_ref[0, p] = jnp.maximum(m0, m1)


@jax.jit
def kernel(proposals, features):
    b, c, hf, wf = features.shape
    n = proposals.shape[1]
    nsteps = n // P
    fx = jnp.float32(hf / IMG_H)
    fy = jnp.float32(wf / IMG_W)

    x0 = jnp.floor(proposals[..., 0] * fx)
    y0 = jnp.floor(proposals[..., 1] * fy)
    w1 = jnp.ceil(proposals[..., 2] * fx) - 1.0
    h1 = jnp.ceil(proposals[..., 3] * fy) - 1.0
    ti = jnp.asarray(_TI)
    tj = jnp.asarray(_TJ)
    sx = jnp.clip(x0[..., None] + ti * w1[..., None], 0.0, hf - 1)
    sy = jnp.clip(y0[..., None] + tj * h1[..., None], 0.0, wf - 1)
    sx = sx.reshape(b, nsteps, 1, NCOL)
    sy = sy.reshape(b, nsteps, 1, NCOL)

    featp = jnp.pad(features.astype(jnp.bfloat16),
                    ((0, 0), (0, 0), (0, 0), (0, WFP - wf)))
    feat2 = featp.reshape(b, c, hf * WFP)

    out = pl.pallas_call(
        functools.partial(_roi_kernel, hf=hf, wf=wf),
        grid=(b, nsteps),
        in_specs=[
            pl.BlockSpec((1, 1, 1, NCOL), lambda bb, nn: (bb, nn, 0, 0)),
            pl.BlockSpec((1, 1, 1, NCOL), lambda bb, nn: (bb, nn, 0, 0)),
            pl.BlockSpec((1, c, hf * WFP), lambda bb, nn: (bb, 0, 0)),
        ],
        out_specs=pl.BlockSpec((1, P, c, NPIX), lambda bb, nn: (bb, nn, 0, 0)),
        out_shape=jax.ShapeDtypeStruct((b, n, c, NPIX), jnp.bfloat16),
        compiler_params=pltpu.CompilerParams(
            dimension_semantics=("arbitrary", "arbitrary"),
            vmem_limit_bytes=100 * 1024 * 1024,
        ),
    )(sx, sy, feat2)
    return out.reshape(b, n, c, OUT // 2, OUT // 2).astype(jnp.float32)
